# baseline (speedup 1.0000x reference)
from functools import partial

import jax
import jax.numpy as jnp
from jax import lax
from jax.experimental import pallas as pl
from jax.experimental.pallas import tpu as pltpu

STRIDE = 8


def _enc_dir_kernel(x_ref, wih_ref, whh_ref, bih_ref, bhh_ref, out_ref,
                    xg_scr):
    L, H = x_ref.shape
    Hh = whh_ref.shape[1]
    d = pl.program_id(0)

    xg_scr[...] = (jnp.dot(x_ref[...], wih_ref[0],
                           preferred_element_type=jnp.float32) + bih_ref[0])

    whh = whh_ref[0]
    bhh = bhh_ref[0]

    def step(t, h):
        tt = jnp.where(d == 0, t, L - 1 - t)
        gi = xg_scr[pl.ds(tt, 1), :]
        gh = jnp.dot(h, whh, preferred_element_type=jnp.float32) + bhh
        r = jax.nn.sigmoid(gi[:, 0:Hh] + gh[:, 0:Hh])
        z = jax.nn.sigmoid(gi[:, Hh:2 * Hh] + gh[:, Hh:2 * Hh])
        n = jnp.tanh(gi[:, 2 * Hh:3 * Hh] + r * gh[:, 2 * Hh:3 * Hh])
        h = (1.0 - z) * n + z * h
        out_ref[0, pl.ds(tt, 1), :] = h
        return h

    lax.fori_loop(0, L, step, jnp.zeros((1, Hh), jnp.float32), unroll=8)


def _dec_kernel(ef_ref, eb_ref, wih_f_ref, wih_b_ref, whh_ref,
                bih_ref, bhh_ref, w1a_ref, w1b_ref, w2a_ref, w2b_ref,
                loss_ref, xsf_scr, xsb_scr, gi_scr, hs_scr,
                *, num_steps):
    L, Hh = ef_ref.shape
    H = 2 * Hh
    NP = hs_scr.shape[0]
    f32 = jnp.float32

    def gather(k, c):
        xsf_scr[pl.ds(k, 1), :] = ef_ref[pl.ds(k * STRIDE, 1), :]
        xsb_scr[pl.ds(k, 1), :] = eb_ref[pl.ds(k * STRIDE, 1), :]
        return c
    lax.fori_loop(0, num_steps, gather, 0, unroll=8)

    gi_scr[...] = (jnp.dot(xsf_scr[...], wih_f_ref[...],
                           preferred_element_type=f32)
                   + jnp.dot(xsb_scr[...], wih_b_ref[...],
                             preferred_element_type=f32)
                   + bih_ref[...])

    hs_scr[pl.ds(NP - 1, 1), :] = jnp.zeros((1, H), f32)
    whh = whh_ref[...]
    bhh = bhh_ref[...]
    h0 = jnp.concatenate([ef_ref[pl.ds(L - 1, 1), :],
                          eb_ref[pl.ds(L - 1, 1), :]], axis=1)

    def step(s, h):
        gi = gi_scr[pl.ds(s, 1), :]
        gh = jnp.dot(h, whh, preferred_element_type=f32) + bhh
        r = jax.nn.sigmoid(gi[:, 0:H] + gh[:, 0:H])
        z = jax.nn.sigmoid(gi[:, H:2 * H] + gh[:, H:2 * H])
        n = jnp.tanh(gi[:, 2 * H:3 * H] + r * gh[:, 2 * H:3 * H])
        h = (1.0 - z) * n + z * h
        hs_scr[pl.ds(s, 1), :] = h
        return h

    lax.fori_loop(0, num_steps, step, h0, unroll=8)

    dn = (((1,), (1,)), ((), ()))
    hs = hs_scr[...]
    ef = ef_ref[...]
    eb = eb_ref[...]
    a_f = jnp.dot(hs, w1a_ref[...], preferred_element_type=f32)
    a_b = jnp.dot(hs, w1b_ref[...], preferred_element_type=f32)
    scores = (lax.dot_general(a_f, ef, dn, preferred_element_type=f32)
              + lax.dot_general(a_b, eb, dn, preferred_element_type=f32)
              + lax.dot_general(w2a_ref[...], ef, dn,
                                preferred_element_type=f32)
              + lax.dot_general(w2b_ref[...], eb, dn,
                                preferred_element_type=f32))

    zc = scores - scores
    law = zc - jnp.log(jnp.exp(zc))

    row = lax.broadcasted_iota(jnp.int32, (NP, L), 0)
    col = lax.broadcasted_iota(jnp.int32, (NP, L), 1)
    pick = (col == (row + 1) * STRIDE) & (row < num_steps)
    contrib = jnp.where(pick, law, 0.0)
    loss_ref[...] = -jnp.sum(jnp.sum(contrib, axis=1, keepdims=True),
                             axis=0, keepdims=True)


def kernel(word_embeddings, enc_wih_t, enc_whh_f_t, enc_whh_b_t, enc_bih,
           enc_bhh_f, enc_bhh_b, dec_wih_t, dec_whh_t, dec_bih, dec_bhh,
           w1, w2):
    L, H = word_embeddings.shape
    Hh = enc_whh_f_t.shape[0]
    G = 3 * Hh

    wih_stack = jnp.stack([enc_wih_t[:, :G], enc_wih_t[:, G:]])
    whh_stack = jnp.stack([enc_whh_f_t, enc_whh_b_t])
    bih_stack = jnp.stack([enc_bih[:, :G], enc_bih[:, G:]])
    bhh_stack = jnp.stack([enc_bhh_f, enc_bhh_b])

    enc = pl.pallas_call(
        _enc_dir_kernel,
        out_shape=jax.ShapeDtypeStruct((2, L, Hh), jnp.float32),
        grid=(2,),
        in_specs=[pl.BlockSpec((L, H), lambda i: (0, 0)),
                  pl.BlockSpec((1, H, G), lambda i: (i, 0, 0)),
                  pl.BlockSpec((1, Hh, G), lambda i: (i, 0, 0)),
                  pl.BlockSpec((1, 1, G), lambda i: (i, 0, 0)),
                  pl.BlockSpec((1, 1, G), lambda i: (i, 0, 0))],
        out_specs=pl.BlockSpec((1, L, Hh), lambda i: (i, 0, 0)),
        scratch_shapes=[pltpu.VMEM((L, G), jnp.float32)],
        compiler_params=pltpu.CompilerParams(
            dimension_semantics=("parallel",)),
    )(word_embeddings, wih_stack, whh_stack, bih_stack, bhh_stack)

    num_steps = L // STRIDE - 1
    NP = num_steps + 1

    full2 = lambda i: (0, 0)
    loss = pl.pallas_call(
        partial(_dec_kernel, num_steps=num_steps),
        out_shape=jax.ShapeDtypeStruct((1, 1), jnp.float32),
        grid=(1,),
        in_specs=[pl.BlockSpec((L, Hh), full2),
                  pl.BlockSpec((L, Hh), full2),
                  pl.BlockSpec((Hh, 3 * H), full2),
                  pl.BlockSpec((Hh, 3 * H), full2),
                  pl.BlockSpec((H, 3 * H), full2),
                  pl.BlockSpec((1, 3 * H), full2),
                  pl.BlockSpec((1, 3 * H), full2),
                  pl.BlockSpec((H, Hh), full2),
                  pl.BlockSpec((H, Hh), full2),
                  pl.BlockSpec((1, Hh), full2),
                  pl.BlockSpec((1, Hh), full2)],
        out_specs=pl.BlockSpec((1, 1), full2),
        scratch_shapes=[pltpu.VMEM((NP, Hh), jnp.float32),
                        pltpu.VMEM((NP, Hh), jnp.float32),
                        pltpu.VMEM((NP, 3 * H), jnp.float32),
                        pltpu.VMEM((NP, H), jnp.float32)],
        compiler_params=pltpu.CompilerParams(
            dimension_semantics=("arbitrary",)),
    )(enc[0], enc[1], dec_wih_t[:Hh], dec_wih_t[Hh:], dec_whh_t,
      dec_bih, dec_bhh, w1[:, :Hh], w1[:, Hh:], w2[:, :Hh], w2[:, Hh:])
    return loss.reshape(1)


# speedup vs baseline: 1.3513x; 1.3513x over previous
from functools import partial

import jax
import jax.numpy as jnp
from jax import lax
from jax.experimental import pallas as pl
from jax.experimental.pallas import tpu as pltpu

STRIDE = 8


def _segmenter_kernel(x_ref, ewih_ref, ewhh_f_ref, ewhh_b_ref,
                      ebih_ref, ebhh_f_ref, ebhh_b_ref,
                      dwih_ref, dwhh_ref, dbih_ref, dbhh_ref,
                      w1_ref, w2_ref,
                      loss_ref,
                      xg_scr, enc_scr, xs_scr, gi_scr, hs_scr,
                      *, num_steps):
    L, H = x_ref.shape
    Hh = ewhh_f_ref.shape[0]
    G = 3 * Hh
    NP = hs_scr.shape[0]
    f32 = jnp.float32
    bf16 = jnp.bfloat16

    xg_scr[...] = (jnp.dot(x_ref[...], ewih_ref[...],
                           preferred_element_type=f32) + ebih_ref[...])

    whh_f = ewhh_f_ref[...]
    whh_b = ewhh_b_ref[...]
    bhh_f = ebhh_f_ref[...]
    bhh_b = ebhh_b_ref[...]

    def gru_cell(gi, gh, h, n):
        r = jax.nn.sigmoid(gi[:, 0:n] + gh[:, 0:n])
        z = jax.nn.sigmoid(gi[:, n:2 * n] + gh[:, n:2 * n])
        nn = jnp.tanh(gi[:, 2 * n:3 * n] + r * gh[:, 2 * n:3 * n])
        return (1.0 - z) * nn + z * h

    def estep(t, carry):
        h_f, h_b = carry
        tb = L - 1 - t
        gi_f = xg_scr[pl.ds(t, 1), 0:G]
        gi_b = xg_scr[pl.ds(tb, 1), G:2 * G]
        gh_f = jnp.dot(h_f.astype(bf16), whh_f,
                       preferred_element_type=f32) + bhh_f
        gh_b = jnp.dot(h_b.astype(bf16), whh_b,
                       preferred_element_type=f32) + bhh_b
        h_f = gru_cell(gi_f, gh_f, h_f, Hh)
        h_b = gru_cell(gi_b, gh_b, h_b, Hh)
        enc_scr[pl.ds(t, 1), 0:Hh] = h_f
        enc_scr[pl.ds(tb, 1), Hh:2 * Hh] = h_b
        return (h_f, h_b)

    h0 = jnp.zeros((1, Hh), f32)
    lax.fori_loop(0, L, estep, (h0, h0), unroll=8)

    def gather(k, c):
        xs_scr[pl.ds(k, 1), :] = enc_scr[pl.ds(k * STRIDE, 1), :]
        return c
    lax.fori_loop(0, num_steps, gather, 0, unroll=8)
    xs_scr[pl.ds(NP - 1, 1), :] = jnp.zeros((1, H), f32)

    gi_scr[...] = (jnp.dot(xs_scr[...].astype(bf16), dwih_ref[...],
                           preferred_element_type=f32) + dbih_ref[...])

    dwhh = dwhh_ref[...]
    dbhh = dbhh_ref[...]

    def dstep(s, h):
        gi = gi_scr[pl.ds(s, 1), :]
        gh = jnp.dot(h.astype(bf16), dwhh, preferred_element_type=f32) + dbhh
        h = gru_cell(gi, gh, h, H)
        hs_scr[pl.ds(s, 1), :] = h
        return h

    hs_scr[pl.ds(NP - 1, 1), :] = jnp.zeros((1, H), f32)
    lax.fori_loop(0, num_steps, dstep, enc_scr[pl.ds(L - 1, 1), :], unroll=8)

    dn = (((1,), (1,)), ((), ()))
    enc_bf = enc_scr[...].astype(bf16)
    a = jnp.dot(hs_scr[...].astype(bf16), w1_ref[...],
                preferred_element_type=f32)
    scores = (lax.dot_general(a.astype(bf16), enc_bf, dn,
                              preferred_element_type=f32)
              + lax.dot_general(w2_ref[...], enc_bf, dn,
                                preferred_element_type=f32))

    zc = scores - scores
    law = zc - jnp.log(jnp.exp(zc))

    row = lax.broadcasted_iota(jnp.int32, (NP, L), 0)
    col = lax.broadcasted_iota(jnp.int32, (NP, L), 1)
    pick = (col == (row + 1) * STRIDE) & (row < num_steps)
    contrib = jnp.where(pick, law, 0.0)
    loss_ref[...] = -jnp.sum(jnp.sum(contrib, axis=1, keepdims=True),
                             axis=0, keepdims=True)


def kernel(word_embeddings, enc_wih_t, enc_whh_f_t, enc_whh_b_t, enc_bih,
           enc_bhh_f, enc_bhh_b, dec_wih_t, dec_whh_t, dec_bih, dec_bhh,
           w1, w2):
    L, H = word_embeddings.shape
    Hh = enc_whh_f_t.shape[0]
    G = 3 * Hh
    bf16 = jnp.bfloat16
    f32 = jnp.float32

    num_steps = L // STRIDE - 1
    NP = num_steps + 1

    full2 = lambda i: (0, 0)
    loss = pl.pallas_call(
        partial(_segmenter_kernel, num_steps=num_steps),
        out_shape=jax.ShapeDtypeStruct((1, 1), f32),
        grid=(1,),
        in_specs=[pl.BlockSpec((L, H), full2),
                  pl.BlockSpec((H, 2 * G), full2),
                  pl.BlockSpec((Hh, G), full2),
                  pl.BlockSpec((Hh, G), full2),
                  pl.BlockSpec((1, 2 * G), full2),
                  pl.BlockSpec((1, G), full2),
                  pl.BlockSpec((1, G), full2),
                  pl.BlockSpec((H, 3 * H), full2),
                  pl.BlockSpec((H, 3 * H), full2),
                  pl.BlockSpec((1, 3 * H), full2),
                  pl.BlockSpec((1, 3 * H), full2),
                  pl.BlockSpec((H, H), full2),
                  pl.BlockSpec((1, H), full2)],
        out_specs=pl.BlockSpec((1, 1), full2),
        scratch_shapes=[pltpu.VMEM((L, 2 * G), f32),
                        pltpu.VMEM((L, H), f32),
                        pltpu.VMEM((NP, H), f32),
                        pltpu.VMEM((NP, 3 * H), f32),
                        pltpu.VMEM((NP, H), f32)],
        compiler_params=pltpu.CompilerParams(
            dimension_semantics=("arbitrary",)),
    )(word_embeddings.astype(bf16), enc_wih_t.astype(bf16),
      enc_whh_f_t.astype(bf16), enc_whh_b_t.astype(bf16),
      enc_bih, enc_bhh_f, enc_bhh_b,
      dec_wih_t.astype(bf16), dec_whh_t.astype(bf16), dec_bih, dec_bhh,
      w1.astype(bf16), w2.astype(bf16))
    return loss.reshape(1)
